# Initial kernel scaffold; baseline (speedup 1.0000x reference)
import jax
import jax.numpy as jnp
from jax.experimental import pallas as pl


def kernel(wd, bd, wh, wx, bg, attn_w, eo_w_src, eo_w_e, eo_b, nu_w_self, nu_w_h, nu_b, fc_w, fc_b, node_features, edge_features, delta_t, edge_len, src_idx, layer_nid):
    raise NotImplementedError("write your pallas kernel here")


# speedup vs baseline: 1.3207x; 1.3207x over previous
from functools import partial

import jax
import jax.numpy as jnp
from jax.experimental import pallas as pl
from jax.experimental.pallas import tpu as pltpu


def _edge_kernel(e_ref, dt_ref, valid_ref, hsrc_ref,
                 wd_ref, bd_ref, wh_ref, wx_ref, bg_ref,
                 attnw_ref, eosrc_ref, eoe_ref, eob_ref,
                 m_ref, a_ref, *, hidden, t_steps):
    H = hidden
    T = t_steps
    TE = dt_ref.shape[0]
    f32 = jnp.float32
    bf16 = jnp.bfloat16

    e2 = e_ref[...].reshape(T * TE, e_ref.shape[2])
    xg = jnp.dot(e2, wx_ref[...], preferred_element_type=f32)
    xg = xg + bg_ref[...].astype(f32)
    hsm = jnp.dot(hsrc_ref[...], eosrc_ref[...], preferred_element_type=f32)
    hsm = hsm + eob_ref[...].astype(f32)

    dtm = dt_ref[...] - 1.0
    valid = valid_ref[...]
    wd = wd_ref[...]
    wh = wh_ref[...]
    bd = bd_ref[...].astype(f32)

    h = jnp.zeros((TE, 2 * H), f32)
    c = jnp.zeros((TE, 2 * H), f32)
    h_last = jnp.zeros((TE, 2 * H), f32)

    for s in range(T):
        c_s = jnp.tanh(
            jnp.dot(c.astype(bf16), wd, preferred_element_type=f32) + bd)
        c_adj = c + c_s * dtm[:, s:s + 1]
        g = jax.nn.sigmoid(
            jnp.dot(h.astype(bf16), wh, preferred_element_type=f32)
            + xg[s * TE:(s + 1) * TE])
        f = g[:, 0:2 * H]
        i = g[:, 2 * H:4 * H]
        o = g[:, 4 * H:6 * H]
        ct = g[:, 6 * H:8 * H]
        c = f * c_adj + i * ct
        h = o * jnp.tanh(c)
        vs = valid[:, s:s + 1]
        h_last = h_last + vs * (h - h_last)

    e_out = h_last[:, :H]
    a_hid = h_last[:, H:2 * H]

    a = jnp.dot(a_hid.astype(bf16), attnw_ref[...], preferred_element_type=f32)
    a = jnp.where(a > 0.0, a, 0.01 * a)

    m = hsm + jnp.dot(e_out.astype(bf16), eoe_ref[...], preferred_element_type=f32)
    m = jnp.maximum(m, 0.0)

    m_ref[...] = m
    a_ref[...] = a


def _pad_axis(x, size, axis):
    pad = size - x.shape[axis]
    if pad == 0:
        return x
    widths = [(0, 0)] * x.ndim
    widths[axis] = (0, pad)
    return jnp.pad(x, widths)


def _edge_messages(e3, dt2, valid, h_src, fp, *, hidden, tile=512):
    T, E, din_e = e3.shape
    din_n = h_src.shape[1]
    H = hidden
    n_blocks = int(pl.cdiv(E, tile))
    Ep = n_blocks * tile
    e3 = _pad_axis(e3, Ep, 1)
    dt2 = _pad_axis(dt2, Ep, 0)
    valid = _pad_axis(valid, Ep, 0)
    h_src = _pad_axis(h_src, Ep, 0)

    body = partial(_edge_kernel, hidden=H, t_steps=T)
    m, a = pl.pallas_call(
        body,
        out_shape=[jax.ShapeDtypeStruct((Ep, H), jnp.float32),
                   jax.ShapeDtypeStruct((Ep, 1), jnp.float32)],
        grid=(n_blocks,),
        in_specs=[
            pl.BlockSpec((T, tile, din_e), lambda i: (0, i, 0)),
            pl.BlockSpec((tile, T), lambda i: (i, 0)),
            pl.BlockSpec((tile, T), lambda i: (i, 0)),
            pl.BlockSpec((tile, din_n), lambda i: (i, 0)),
            pl.BlockSpec((2 * H, 2 * H), lambda i: (0, 0)),
            pl.BlockSpec((1, 2 * H), lambda i: (0, 0)),
            pl.BlockSpec((2 * H, 8 * H), lambda i: (0, 0)),
            pl.BlockSpec((din_e, 8 * H), lambda i: (0, 0)),
            pl.BlockSpec((1, 8 * H), lambda i: (0, 0)),
            pl.BlockSpec((H, 1), lambda i: (0, 0)),
            pl.BlockSpec((din_n, H), lambda i: (0, 0)),
            pl.BlockSpec((H, H), lambda i: (0, 0)),
            pl.BlockSpec((1, H), lambda i: (0, 0)),
        ],
        out_specs=[
            pl.BlockSpec((tile, H), lambda i: (i, 0)),
            pl.BlockSpec((tile, 1), lambda i: (i, 0)),
        ],
        compiler_params=pltpu.CompilerParams(dimension_semantics=("parallel",)),
    )(e3, dt2, valid, h_src,
      fp["wd"], fp["bd"], fp["wh"], fp["wx"], fp["bg"],
      fp["attn_w"], fp["eo_w_src"], fp["eo_w_e"], fp["eo_b"])
    return m[:E], a[:E]


def _reduce_kernel(a_ref, m_ref, selfh_ref,
                   eosrc_ref, eob_ref,
                   nusrc_ref, nuh_ref, nub_ref,
                   fcw_ref, fcb_ref, o_ref, *, k_deg, hidden):
    K = k_deg
    H = hidden
    f32 = jnp.float32
    z = a_ref[...]
    TD = z.shape[0]

    z = z - jnp.max(z, axis=1, keepdims=True)
    ksum = jnp.zeros((TD, K), f32)
    ssum = jnp.zeros((TD, K), f32)
    for r in range(K):
        zr = z if r == 0 else jnp.roll(z, r, axis=1)
        ge = (zr >= z).astype(f32)
        ksum = ksum + ge
        ssum = ssum + ge * zr
    in_sup = (1.0 + ksum * z > ssum).astype(f32)
    sk = jnp.sum(in_sup, axis=1, keepdims=True)
    sz = jnp.sum(in_sup * z, axis=1, keepdims=True)
    tau = (sz - 1.0) / sk
    alpha = jnp.maximum(z - tau, 0.0)

    m = m_ref[...]
    h_red = jnp.zeros((TD, H), f32)
    for i in range(K):
        h_red = h_red + alpha[:, i:i + 1] * m[:, i * H:(i + 1) * H]

    self_h = selfh_ref[...]
    self_h_tmp = (jnp.dot(self_h, eosrc_ref[...], preferred_element_type=f32)
                  + eob_ref[...])
    hh = (h_red - self_h_tmp).astype(jnp.bfloat16)
    act = (jnp.dot(self_h, nusrc_ref[...], preferred_element_type=f32)
           + jnp.dot(hh, nuh_ref[...], preferred_element_type=f32)
           + nub_ref[...])
    act = jnp.maximum(act, 0.0).astype(jnp.bfloat16)
    out = jnp.dot(act, fcw_ref[...], preferred_element_type=f32) + fcb_ref[...]
    o_ref[...] = out


def _reduce_update(a_mb, m_mb, self_h, fp, *, hidden, k_deg, tile=512):
    ND, K = a_mb.shape
    H = hidden
    din_n = self_h.shape[1]
    C = fp["fc_w"].shape[1]
    n_blocks = int(pl.cdiv(ND, tile))
    NDp = n_blocks * tile
    a_mb = _pad_axis(a_mb, NDp, 0)
    m_mb = _pad_axis(m_mb, NDp, 0)
    self_h = _pad_axis(self_h, NDp, 0)

    body = partial(_reduce_kernel, k_deg=K, hidden=H)
    out = pl.pallas_call(
        body,
        out_shape=jax.ShapeDtypeStruct((NDp, C), jnp.float32),
        grid=(n_blocks,),
        in_specs=[
            pl.BlockSpec((tile, K), lambda i: (i, 0)),
            pl.BlockSpec((tile, K * H), lambda i: (i, 0)),
            pl.BlockSpec((tile, din_n), lambda i: (i, 0)),
            pl.BlockSpec((din_n, H), lambda i: (0, 0)),
            pl.BlockSpec((1, H), lambda i: (0, 0)),
            pl.BlockSpec((din_n, H), lambda i: (0, 0)),
            pl.BlockSpec((H, H), lambda i: (0, 0)),
            pl.BlockSpec((1, H), lambda i: (0, 0)),
            pl.BlockSpec((H, C), lambda i: (0, 0)),
            pl.BlockSpec((1, C), lambda i: (0, 0)),
        ],
        out_specs=pl.BlockSpec((tile, C), lambda i: (i, 0)),
        compiler_params=pltpu.CompilerParams(dimension_semantics=("parallel",)),
    )(a_mb, m_mb, self_h,
      fp["eo_w_src"], fp["eo_b"], fp["nu_w_self"], fp["nu_w_h"], fp["nu_b"],
      fp["fc_w"], fp["fc_b"])
    return out[:ND]


def kernel(wd, bd, wh, wx, bg, attn_w, eo_w_src, eo_w_e, eo_b,
           nu_w_self, nu_w_h, nu_b, fc_w, fc_b,
           node_features, edge_features, delta_t, edge_len, src_idx, layer_nid):
    H = 128
    bf16 = jnp.bfloat16
    n_dst, k = src_idx.shape
    n_edges, t_steps, din_e = edge_features.shape

    h = node_features
    self_h = h[layer_nid]
    h_src = h[src_idx.reshape(-1)].astype(bf16)

    e3 = jnp.transpose(edge_features, (1, 0, 2)).astype(bf16)
    valid = (jnp.arange(t_steps, dtype=jnp.int32)[None, :]
             < edge_len[:, None]).astype(jnp.float32)

    fpA = {
        "wd": wd.astype(bf16), "bd": bd, "wh": wh.astype(bf16),
        "wx": wx.astype(bf16), "bg": bg, "attn_w": attn_w.astype(bf16),
        "eo_w_src": eo_w_src.astype(bf16), "eo_w_e": eo_w_e.astype(bf16),
        "eo_b": eo_b,
    }
    m, a = _edge_messages(e3, delta_t, valid, h_src, fpA, hidden=H)

    a_mb = a.reshape(n_dst, k)
    m_mb = m.reshape(n_dst, k * H)

    fpB = {
        "eo_w_src": eo_w_src, "eo_b": eo_b,
        "nu_w_self": nu_w_self, "nu_w_h": nu_w_h.astype(bf16), "nu_b": nu_b,
        "fc_w": fc_w.astype(bf16), "fc_b": fc_b,
    }
    return _reduce_update(a_mb, m_mb, self_h, fpB, hidden=H, k_deg=k)


# speedup vs baseline: 1.5597x; 1.1810x over previous
from functools import partial

import jax
import jax.numpy as jnp
from jax.experimental import pallas as pl
from jax.experimental.pallas import tpu as pltpu


def _edge_kernel(e_ref, dt_ref, len_ref, hsrc_ref,
                 wd_ref, bd_ref, whx_ref, bg_ref,
                 attnw_ref, eosrc_ref, eoe_ref, eob_ref,
                 m_ref, a_ref, *, hidden, t_steps, din_e):
    H = hidden
    T = t_steps
    D = din_e
    TE = dt_ref.shape[0]
    f32 = jnp.float32
    bf16 = jnp.bfloat16

    hsm = jnp.dot(hsrc_ref[...].astype(bf16), eosrc_ref[...],
                  preferred_element_type=f32) + eob_ref[...]

    dtm = dt_ref[...] - 1.0
    lens = len_ref[...]
    wd = wd_ref[...]
    whx = whx_ref[...]
    bd = bd_ref[...]
    bgh = bg_ref[...]

    h = jnp.zeros((TE, 2 * H), f32)
    c = jnp.zeros((TE, 2 * H), f32)
    h_last = jnp.zeros((TE, 2 * H), f32)

    for s in range(T):
        x_s = e_ref[:, s * D:(s + 1) * D].astype(bf16)
        c_s = jnp.tanh(
            jnp.dot(c.astype(bf16), wd, preferred_element_type=f32) + bd)
        c_adj = c + c_s * dtm[:, s:s + 1]
        hx = jnp.concatenate([h.astype(bf16), x_s], axis=1)
        g = 0.5 * jnp.tanh(
            jnp.dot(hx, whx, preferred_element_type=f32) + bgh) + 0.5
        f = g[:, 0:2 * H]
        i = g[:, 2 * H:4 * H]
        o = g[:, 4 * H:6 * H]
        ct = g[:, 6 * H:8 * H]
        c = f * c_adj + i * ct
        h = o * jnp.tanh(c)
        h_last = jnp.where(lens > s, h, h_last)

    e_out = h_last[:, :H]
    a_hid = h_last[:, H:2 * H]

    a = jnp.dot(a_hid.astype(bf16), attnw_ref[...], preferred_element_type=f32)
    a = jnp.where(a > 0.0, a, 0.01 * a)

    m = hsm + jnp.dot(e_out.astype(bf16), eoe_ref[...], preferred_element_type=f32)
    m = jnp.maximum(m, 0.0)

    m_ref[...] = m
    a_ref[...] = a


def _pad_axis(x, size, axis):
    pad = size - x.shape[axis]
    if pad == 0:
        return x
    widths = [(0, 0)] * x.ndim
    widths[axis] = (0, pad)
    return jnp.pad(x, widths)


def _edge_messages(e2d, dt2, len2, h_src, fp, *, hidden, t_steps, din_e,
                   tile=1000):
    E = e2d.shape[0]
    din_n = h_src.shape[1]
    H = hidden
    n_blocks = int(pl.cdiv(E, tile))
    Ep = n_blocks * tile
    e2d = _pad_axis(e2d, Ep, 0)
    dt2 = _pad_axis(dt2, Ep, 0)
    len2 = _pad_axis(len2, Ep, 0)
    h_src = _pad_axis(h_src, Ep, 0)

    body = partial(_edge_kernel, hidden=H, t_steps=t_steps, din_e=din_e)
    m, a = pl.pallas_call(
        body,
        out_shape=[jax.ShapeDtypeStruct((Ep, H), jnp.float32),
                   jax.ShapeDtypeStruct((Ep, 1), jnp.float32)],
        grid=(n_blocks,),
        in_specs=[
            pl.BlockSpec((tile, t_steps * din_e), lambda i: (i, 0)),
            pl.BlockSpec((tile, t_steps), lambda i: (i, 0)),
            pl.BlockSpec((tile, 1), lambda i: (i, 0)),
            pl.BlockSpec((tile, din_n), lambda i: (i, 0)),
            pl.BlockSpec((2 * H, 2 * H), lambda i: (0, 0)),
            pl.BlockSpec((1, 2 * H), lambda i: (0, 0)),
            pl.BlockSpec((2 * H + din_e, 8 * H), lambda i: (0, 0)),
            pl.BlockSpec((1, 8 * H), lambda i: (0, 0)),
            pl.BlockSpec((H, 1), lambda i: (0, 0)),
            pl.BlockSpec((din_n, H), lambda i: (0, 0)),
            pl.BlockSpec((H, H), lambda i: (0, 0)),
            pl.BlockSpec((1, H), lambda i: (0, 0)),
        ],
        out_specs=[
            pl.BlockSpec((tile, H), lambda i: (i, 0)),
            pl.BlockSpec((tile, 1), lambda i: (i, 0)),
        ],
        compiler_params=pltpu.CompilerParams(dimension_semantics=("parallel",)),
    )(e2d, dt2, len2, h_src,
      fp["wd"], fp["bd"], fp["whx"], fp["bgh"],
      fp["attn_w"], fp["eo_w_src"], fp["eo_w_e"], fp["eo_b"])
    return m[:E], a[:E]


def _reduce_kernel(a_ref, m_ref, selfh_ref,
                   eosrc_ref, eob_ref,
                   nusrc_ref, nuh_ref, nub_ref,
                   fcw_ref, fcb_ref, o_ref, *, k_deg, hidden):
    K = k_deg
    H = hidden
    f32 = jnp.float32
    z = a_ref[...]
    TD = z.shape[0]

    z = z - jnp.max(z, axis=1, keepdims=True)
    ksum = jnp.zeros((TD, K), f32)
    ssum = jnp.zeros((TD, K), f32)
    for r in range(K):
        zr = z if r == 0 else jnp.roll(z, r, axis=1)
        ge = (zr >= z).astype(f32)
        ksum = ksum + ge
        ssum = ssum + ge * zr
    in_sup = (1.0 + ksum * z > ssum).astype(f32)
    sk = jnp.sum(in_sup, axis=1, keepdims=True)
    sz = jnp.sum(in_sup * z, axis=1, keepdims=True)
    tau = (sz - 1.0) / sk
    alpha = jnp.maximum(z - tau, 0.0)

    m = m_ref[...]
    h_red = jnp.zeros((TD, H), f32)
    for i in range(K):
        h_red = h_red + alpha[:, i:i + 1] * m[:, i * H:(i + 1) * H]

    self_h = selfh_ref[...]
    self_h_tmp = (jnp.dot(self_h, eosrc_ref[...], preferred_element_type=f32)
                  + eob_ref[...])
    hh = (h_red - self_h_tmp).astype(jnp.bfloat16)
    act = (jnp.dot(self_h, nusrc_ref[...], preferred_element_type=f32)
           + jnp.dot(hh, nuh_ref[...], preferred_element_type=f32)
           + nub_ref[...])
    act = jnp.maximum(act, 0.0).astype(jnp.bfloat16)
    out = jnp.dot(act, fcw_ref[...], preferred_element_type=f32) + fcb_ref[...]
    o_ref[...] = out


def _reduce_update(a_mb, m_mb, self_h, fp, *, hidden, k_deg, tile=512):
    ND, K = a_mb.shape
    H = hidden
    din_n = self_h.shape[1]
    C = fp["fc_w"].shape[1]
    n_blocks = int(pl.cdiv(ND, tile))
    NDp = n_blocks * tile
    a_mb = _pad_axis(a_mb, NDp, 0)
    m_mb = _pad_axis(m_mb, NDp, 0)
    self_h = _pad_axis(self_h, NDp, 0)

    body = partial(_reduce_kernel, k_deg=K, hidden=H)
    out = pl.pallas_call(
        body,
        out_shape=jax.ShapeDtypeStruct((NDp, C), jnp.float32),
        grid=(n_blocks,),
        in_specs=[
            pl.BlockSpec((tile, K), lambda i: (i, 0)),
            pl.BlockSpec((tile, K * H), lambda i: (i, 0)),
            pl.BlockSpec((tile, din_n), lambda i: (i, 0)),
            pl.BlockSpec((din_n, H), lambda i: (0, 0)),
            pl.BlockSpec((1, H), lambda i: (0, 0)),
            pl.BlockSpec((din_n, H), lambda i: (0, 0)),
            pl.BlockSpec((H, H), lambda i: (0, 0)),
            pl.BlockSpec((1, H), lambda i: (0, 0)),
            pl.BlockSpec((H, C), lambda i: (0, 0)),
            pl.BlockSpec((1, C), lambda i: (0, 0)),
        ],
        out_specs=pl.BlockSpec((tile, C), lambda i: (i, 0)),
        compiler_params=pltpu.CompilerParams(dimension_semantics=("parallel",)),
    )(a_mb, m_mb, self_h,
      fp["eo_w_src"], fp["eo_b"], fp["nu_w_self"], fp["nu_w_h"], fp["nu_b"],
      fp["fc_w"], fp["fc_b"])
    return out[:ND]


def kernel(wd, bd, wh, wx, bg, attn_w, eo_w_src, eo_w_e, eo_b,
           nu_w_self, nu_w_h, nu_b, fc_w, fc_b,
           node_features, edge_features, delta_t, edge_len, src_idx, layer_nid):
    H = 128
    bf16 = jnp.bfloat16
    n_dst, k = src_idx.shape
    n_edges, t_steps, din_e = edge_features.shape

    h = node_features
    self_h = h[layer_nid]
    h_src = h[src_idx.reshape(-1)]

    e2d = edge_features.reshape(n_edges, t_steps * din_e)
    len2 = edge_len.reshape(n_edges, 1)

    fpA = {
        "wd": wd.astype(bf16), "bd": bd,
        "whx": (0.5 * jnp.concatenate([wh, wx], axis=0)).astype(bf16),
        "bgh": 0.5 * bg,
        "attn_w": attn_w.astype(bf16),
        "eo_w_src": eo_w_src.astype(bf16), "eo_w_e": eo_w_e.astype(bf16),
        "eo_b": eo_b,
    }
    m, a = _edge_messages(e2d, delta_t, len2, h_src, fpA,
                          hidden=H, t_steps=t_steps, din_e=din_e)

    a_mb = a.reshape(n_dst, k)
    m_mb = m.reshape(n_dst, k * H)

    fpB = {
        "eo_w_src": eo_w_src, "eo_b": eo_b,
        "nu_w_self": nu_w_self, "nu_w_h": nu_w_h.astype(bf16), "nu_b": nu_b,
        "fc_w": fc_w.astype(bf16), "fc_b": fc_b,
    }
    return _reduce_update(a_mb, m_mb, self_h, fpB, hidden=H, k_deg=k)
